# baseline (speedup 1.0000x reference)
import functools

import jax
import jax.numpy as jnp
from jax import lax
from jax.experimental import pallas as pl
from jax.experimental.pallas import tpu as pltpu

_ML = 2.9086
_SL = 1.898
_SCALE_MR = 500.0
_INV_SCALE_MR = 1.0 / _SCALE_MR
_LANES = 128

_COL_H = 0
_COL_C = 1
_COL_L = 2
_COL_LC = 3
_COL_BP = 4
_COL_IB = 5
_COL_OO = 6
_COL_OL = 7
_COL_OLC = 8
_COL_F = 9
_COL_STD = 10
_COL_OV = 11

(_P_HOO1, _P_G1, _P_KOOH, _P_AOOH, _P_SIG, _P_EXP, _P_THR, _P_STD) = range(8)
_N_PARAMS = 8


def _round_up(x, m):
    return (x + m - 1) // m * m


def _rnn_kernel(u1_ref, u2_ref, ol_ref, p_ref, out_ref, c_state, *,
                batch, seq_len, time_lag, block_rows):
    blk = pl.program_id(0)
    base = blk * block_rows

    @pl.when(blk == 0)
    def _():
        c_state[...] = jnp.zeros_like(c_state)

    hoo1 = p_ref[_P_HOO1]
    g1 = p_ref[_P_G1]
    koo_h = p_ref[_P_KOOH]
    aoo_h = p_ref[_P_AOOH]
    sig = p_ref[_P_SIG]
    exp_yrm = p_ref[_P_EXP]
    thr = p_ref[_P_THR]
    obs_std = p_ref[_P_STD]

    shape = (1, _LANES)
    lane = lax.broadcasted_iota(jnp.int32, shape, 1)
    _used = (_COL_H, _COL_C, _COL_L, _COL_LC, _COL_OO, _COL_OL,
             _COL_OLC, _COL_F, _COL_STD, _COL_OV)
    onehot = {j: (lane == j).astype(jnp.float32) for j in _used}

    def step(c0, u1, u2, ol):
        cpos = c0 > 0.0
        t = jnp.tanh(koo_h + c0 * aoo_h)
        r = pl.reciprocal(c0, approx=True)
        a1 = hoo1 * t
        olc = jnp.where(cpos, jnp.minimum(ol, u2 * r), ol)
        f = (g1 - a1) - olc
        d = c0 - thr
        ad = jnp.abs(d)
        c1 = (f * c0 + u1) - jnp.minimum(sig * d, f * ad)
        return c1, a1, olc, f

    r_start = jnp.maximum(0, time_lag - base)
    r_end = jnp.minimum(block_rows, batch - base)

    @pl.when(r_start > 0)
    def _():
        out_ref[...] = jnp.zeros(out_ref.shape, out_ref.dtype)

    def row_body(r, c0):
        row_off = (base + r) * seq_len
        c = c0
        for t in range(seq_len - 1):
            c = step(c, u1_ref[row_off + t], u2_ref[row_off + t],
                     ol_ref[row_off + t])[0]
        idx = row_off + seq_len - 1
        ol = ol_ref[idx]
        c_new, a1, olc, f = step(c, u1_ref[idx], u2_ref[idx], ol)
        oo = hoo1 + a1
        sgn = jnp.sign(c * _INV_SCALE_MR - exp_yrm)
        ov = jnp.minimum(sig * sgn, f)
        packed = ((oo * c) * onehot[_COL_H]
                  + c * onehot[_COL_C]
                  + (ol * c) * onehot[_COL_L]
                  + (olc * c) * onehot[_COL_LC]
                  + oo * onehot[_COL_OO]
                  + ol * onehot[_COL_OL]
                  + olc * onehot[_COL_OLC]
                  + f * onehot[_COL_F]
                  + obs_std * onehot[_COL_STD]
                  + ov * onehot[_COL_OV])
        out_ref[pl.ds(r, 1), :] = packed
        return c_new

    c_final = lax.fori_loop(r_start, r_end, row_body, c_state[...])
    c_state[...] = c_final


def _forward(x, y_obs, params, p_mean, p_std, *, time_lag, spin_len,
             train_len):
    batch, seq, _ = x.shape
    x = x.astype(jnp.float32)
    u1 = x[:, :, 0].reshape(-1)
    u2 = x[:, :, 1].reshape(-1)

    f32 = lambda v: jnp.asarray(v, jnp.float32).reshape(())
    w_r_yom = f32(params['weight_r_yom'])
    w_r_ylm = f32(params['weight_r_ylm'])
    w_r_yfm = f32(params['weight_r_yfm'])
    w_r_yvm = f32(params['weight_r_yvm'])
    b0_yom = f32(params['bias_b0_yom'])
    w_b1_yom = f32(params['weight_b1_yom'])
    b0_ylm = f32(params['bias_b0_ylm'])
    w_b2_ylm = f32(params['weight_b2_ylm'])
    b0_yrm = f32(params['bias_b0_yrm'])
    mo = f32(p_mean)
    so = f32(p_std)

    e_o, e_l, e_f = jnp.exp(w_r_yom), jnp.exp(w_r_ylm), jnp.exp(w_r_yfm)
    denom = e_o + e_l + e_f
    oo1 = e_o / denom
    ol1 = e_l / denom
    sig_yvm = jax.nn.sigmoid(w_r_yvm)
    exp_yrm = jnp.exp(b0_yrm)
    thr = exp_yrm * jnp.float32(_SCALE_MR)
    a_oo = w_b1_yom / so
    k_oo = b0_yom - mo * a_oo
    a_ol = w_b2_ylm / jnp.float32(_SL)
    k_ol = b0_ylm - jnp.float32(_ML) * a_ol
    ol_all = (ol1 * jax.nn.sigmoid(k_ol + u2 * a_ol)).astype(jnp.float32)
    obs_std = jnp.std(y_obs[spin_len:train_len].astype(jnp.float32), ddof=1)

    hoo1 = 0.5 * oo1
    p_vec = jnp.stack([hoo1, 1.0 - hoo1, 0.5 * k_oo, 0.5 * a_oo, sig_yvm,
                       exp_yrm, thr, obs_std]).astype(jnp.float32)

    block_rows = min(256, _round_up(batch, 8))
    grid = pl.cdiv(batch, block_rows)

    _kernel_fn = functools.partial(_rnn_kernel, batch=batch, seq_len=seq,
                                   time_lag=time_lag, block_rows=block_rows)

    out = pl.pallas_call(
        _kernel_fn,
        out_shape=jax.ShapeDtypeStruct((batch, _LANES), jnp.float32),
        grid_spec=pltpu.PrefetchScalarGridSpec(
            num_scalar_prefetch=0,
            grid=(grid,),
            in_specs=[
                pl.BlockSpec(memory_space=pltpu.MemorySpace.SMEM),
                pl.BlockSpec(memory_space=pltpu.MemorySpace.SMEM),
                pl.BlockSpec(memory_space=pltpu.MemorySpace.SMEM),
                pl.BlockSpec(memory_space=pltpu.MemorySpace.SMEM),
            ],
            out_specs=pl.BlockSpec((block_rows, _LANES), lambda i: (i, 0)),
            scratch_shapes=[pltpu.VMEM((1, _LANES), jnp.float32)],
        ),
        compiler_params=pltpu.CompilerParams(
            dimension_semantics=("arbitrary",)),
    )(u1, u2, ol_all, p_vec)

    col = lambda j: out[:, j:j + 1]
    h_n = col(_COL_H)
    obs_std_col = col(_COL_STD)
    h_nout = jnp.concatenate([h_n, obs_std_col], axis=1)
    return (h_n, col(_COL_C), col(_COL_L), col(_COL_LC), col(_COL_BP),
            col(_COL_IB), col(_COL_OO), col(_COL_OL), col(_COL_OLC),
            col(_COL_F), h_nout, obs_std_col, col(_COL_OV))


def kernel(x, y_obs, weight_r_yom, weight_r_ylm, weight_r_yfm, weight_r_yvm,
           bias_b0_yom, weight_b1_yom, bias_b0_ylm, weight_b2_ylm,
           bias_b0_yrm, p_mean, p_std):
    params = {
        'weight_r_yom': weight_r_yom,
        'weight_r_ylm': weight_r_ylm,
        'weight_r_yfm': weight_r_yfm,
        'weight_r_yvm': weight_r_yvm,
        'bias_b0_yom': bias_b0_yom,
        'weight_b1_yom': weight_b1_yom,
        'bias_b0_ylm': bias_b0_ylm,
        'weight_b2_ylm': weight_b2_ylm,
        'bias_b0_yrm': bias_b0_yrm,
    }
    return _forward(x, y_obs, params, p_mean, p_std,
                    time_lag=128, spin_len=128, train_len=4096)


# speedup vs baseline: 4.6746x; 4.6746x over previous
import functools

import jax
import jax.numpy as jnp
from jax import lax
from jax.experimental import pallas as pl
from jax.experimental.pallas import tpu as pltpu

_ML = 2.9086
_SL = 1.898
_SCALE_MR = 500.0
_INV_SCALE_MR = 1.0 / _SCALE_MR
_LANES = 128

_COL_H = 0
_COL_C = 1
_COL_L = 2
_COL_LC = 3
_COL_BP = 4
_COL_IB = 5
_COL_OO = 6
_COL_OL = 7
_COL_OLC = 8
_COL_F = 9
_COL_STD = 10
_COL_OV = 11

(_P_HOO1, _P_G1, _P_KOOH, _P_AOOH, _P_SIG, _P_EXP, _P_THR, _P_STD) = range(8)
_N_PARAMS = 8


def _round_up(x, m):
    return (x + m - 1) // m * m


def _rnn_kernel(u1_ref, u2_ref, ol_ref, p_ref, out_ref, c_state, *,
                batch, seq_len, time_lag, block_rows):
    blk = pl.program_id(0)
    base = blk * block_rows

    @pl.when(blk == 0)
    def _():
        c_state[...] = jnp.zeros_like(c_state)

    shape = (1, _LANES)

    def splat(j):
        return jnp.full(shape, p_ref[j], dtype=jnp.float32)

    hoo1 = splat(_P_HOO1)
    g1 = splat(_P_G1)
    koo_h = splat(_P_KOOH)
    aoo_h = splat(_P_AOOH)
    sig = splat(_P_SIG)
    exp_yrm = splat(_P_EXP)
    thr = splat(_P_THR)
    obs_std = splat(_P_STD)
    lane = lax.broadcasted_iota(jnp.int32, shape, 1)
    _used = (_COL_H, _COL_C, _COL_L, _COL_LC, _COL_OO, _COL_OL,
             _COL_OLC, _COL_F, _COL_STD, _COL_OV)
    onehot = {j: (lane == j).astype(jnp.float32) for j in _used}

    def step(c0, u1, u2, ol):
        cpos = c0 > 0.0
        t = jnp.tanh(koo_h + c0 * aoo_h)
        r = pl.reciprocal(c0, approx=True)
        a1 = hoo1 * t
        olc = jnp.where(cpos, jnp.minimum(ol, u2 * r), ol)
        f = (g1 - a1) - olc
        d = c0 - thr
        ad = jnp.abs(d)
        c1 = (f * c0 + u1) - jnp.minimum(sig * d, f * ad)
        return c1, a1, olc, f

    r_start = jnp.maximum(0, time_lag - base)
    r_end = jnp.minimum(block_rows, batch - base)

    @pl.when(r_start > 0)
    def _():
        out_ref[...] = jnp.zeros(out_ref.shape, out_ref.dtype)

    def row_body(r, c0):
        row_off = (base + r) * seq_len
        c = c0
        for t in range(seq_len - 1):
            c = step(c, u1_ref[row_off + t], u2_ref[row_off + t],
                     ol_ref[row_off + t])[0]
        idx = row_off + seq_len - 1
        ol = ol_ref[idx]
        c_new, a1, olc, f = step(c, u1_ref[idx], u2_ref[idx], ol)
        oo = hoo1 + a1
        sgn = jnp.sign(c * _INV_SCALE_MR - exp_yrm)
        ov = jnp.minimum(sig * sgn, f)
        packed = ((oo * c) * onehot[_COL_H]
                  + c * onehot[_COL_C]
                  + (ol * c) * onehot[_COL_L]
                  + (olc * c) * onehot[_COL_LC]
                  + oo * onehot[_COL_OO]
                  + ol * onehot[_COL_OL]
                  + olc * onehot[_COL_OLC]
                  + f * onehot[_COL_F]
                  + obs_std * onehot[_COL_STD]
                  + ov * onehot[_COL_OV])
        out_ref[pl.ds(r, 1), :] = packed
        return c_new

    c_final = lax.fori_loop(r_start, r_end, row_body, c_state[...])
    c_state[...] = c_final


def _forward(x, y_obs, params, p_mean, p_std, *, time_lag, spin_len,
             train_len):
    batch, seq, _ = x.shape
    x = x.astype(jnp.float32)
    u1 = x[:, :, 0].reshape(-1)
    u2 = x[:, :, 1].reshape(-1)

    f32 = lambda v: jnp.asarray(v, jnp.float32).reshape(())
    w_r_yom = f32(params['weight_r_yom'])
    w_r_ylm = f32(params['weight_r_ylm'])
    w_r_yfm = f32(params['weight_r_yfm'])
    w_r_yvm = f32(params['weight_r_yvm'])
    b0_yom = f32(params['bias_b0_yom'])
    w_b1_yom = f32(params['weight_b1_yom'])
    b0_ylm = f32(params['bias_b0_ylm'])
    w_b2_ylm = f32(params['weight_b2_ylm'])
    b0_yrm = f32(params['bias_b0_yrm'])
    mo = f32(p_mean)
    so = f32(p_std)

    e_o, e_l, e_f = jnp.exp(w_r_yom), jnp.exp(w_r_ylm), jnp.exp(w_r_yfm)
    denom = e_o + e_l + e_f
    oo1 = e_o / denom
    ol1 = e_l / denom
    sig_yvm = jax.nn.sigmoid(w_r_yvm)
    exp_yrm = jnp.exp(b0_yrm)
    thr = exp_yrm * jnp.float32(_SCALE_MR)
    a_oo = w_b1_yom / so
    k_oo = b0_yom - mo * a_oo
    a_ol = w_b2_ylm / jnp.float32(_SL)
    k_ol = b0_ylm - jnp.float32(_ML) * a_ol
    ol_all = (ol1 * jax.nn.sigmoid(k_ol + u2 * a_ol)).astype(jnp.float32)
    obs_std = jnp.std(y_obs[spin_len:train_len].astype(jnp.float32), ddof=1)

    hoo1 = 0.5 * oo1
    p_vec = jnp.stack([hoo1, 1.0 - hoo1, 0.5 * k_oo, 0.5 * a_oo, sig_yvm,
                       exp_yrm, thr, obs_std]).astype(jnp.float32)

    block_rows = min(256, _round_up(batch, 8))
    grid = pl.cdiv(batch, block_rows)

    _kernel_fn = functools.partial(_rnn_kernel, batch=batch, seq_len=seq,
                                   time_lag=time_lag, block_rows=block_rows)

    out = pl.pallas_call(
        _kernel_fn,
        out_shape=jax.ShapeDtypeStruct((batch, _LANES), jnp.float32),
        grid_spec=pltpu.PrefetchScalarGridSpec(
            num_scalar_prefetch=0,
            grid=(grid,),
            in_specs=[
                pl.BlockSpec(memory_space=pltpu.MemorySpace.SMEM),
                pl.BlockSpec(memory_space=pltpu.MemorySpace.SMEM),
                pl.BlockSpec(memory_space=pltpu.MemorySpace.SMEM),
                pl.BlockSpec(memory_space=pltpu.MemorySpace.SMEM),
            ],
            out_specs=pl.BlockSpec((block_rows, _LANES), lambda i: (i, 0)),
            scratch_shapes=[pltpu.VMEM((1, _LANES), jnp.float32)],
        ),
        compiler_params=pltpu.CompilerParams(
            dimension_semantics=("arbitrary",)),
    )(u1, u2, ol_all, p_vec)

    col = lambda j: out[:, j:j + 1]
    h_n = col(_COL_H)
    obs_std_col = col(_COL_STD)
    h_nout = jnp.concatenate([h_n, obs_std_col], axis=1)
    return (h_n, col(_COL_C), col(_COL_L), col(_COL_LC), col(_COL_BP),
            col(_COL_IB), col(_COL_OO), col(_COL_OL), col(_COL_OLC),
            col(_COL_F), h_nout, obs_std_col, col(_COL_OV))


def kernel(x, y_obs, weight_r_yom, weight_r_ylm, weight_r_yfm, weight_r_yvm,
           bias_b0_yom, weight_b1_yom, bias_b0_ylm, weight_b2_ylm,
           bias_b0_yrm, p_mean, p_std):
    params = {
        'weight_r_yom': weight_r_yom,
        'weight_r_ylm': weight_r_ylm,
        'weight_r_yfm': weight_r_yfm,
        'weight_r_yvm': weight_r_yvm,
        'bias_b0_yom': bias_b0_yom,
        'weight_b1_yom': weight_b1_yom,
        'bias_b0_ylm': bias_b0_ylm,
        'weight_b2_ylm': weight_b2_ylm,
        'bias_b0_yrm': bias_b0_yrm,
    }
    return _forward(x, y_obs, params, p_mean, p_std,
                    time_lag=128, spin_len=128, train_len=4096)


# speedup vs baseline: 5.0632x; 1.0831x over previous
import functools

import jax
import jax.numpy as jnp
from jax import lax
from jax.experimental import pallas as pl
from jax.experimental.pallas import tpu as pltpu

_ML = 2.9086
_SL = 1.898
_SCALE_MR = 500.0
_INV_SCALE_MR = 1.0 / _SCALE_MR
_LANES = 128

_COL_H = 0
_COL_C = 1
_COL_L = 2
_COL_LC = 3
_COL_BP = 4
_COL_IB = 5
_COL_OO = 6
_COL_OL = 7
_COL_OLC = 8
_COL_F = 9
_COL_STD = 10
_COL_OV = 11

(_P_HOO1, _P_G1, _P_KOOH, _P_AOOH, _P_SIG, _P_EXP, _P_THR, _P_STD) = range(8)
_N_PARAMS = 8


def _round_up(x, m):
    return (x + m - 1) // m * m


def _rnn_kernel(u1_ref, u2_ref, ol_ref, p_ref, out_ref, c_state, *,
                batch, seq_len, time_lag, block_rows):
    blk = pl.program_id(0)
    base = blk * block_rows

    @pl.when(blk == 0)
    def _():
        c_state[...] = jnp.zeros_like(c_state)

    shape = (1, _LANES)

    def splat(j):
        return jnp.full(shape, p_ref[j], dtype=jnp.float32)

    hoo1 = splat(_P_HOO1)
    g1 = splat(_P_G1)
    koo_h = splat(_P_KOOH)
    aoo_h = splat(_P_AOOH)
    sig = splat(_P_SIG)
    exp_yrm = splat(_P_EXP)
    thr = splat(_P_THR)
    obs_std = splat(_P_STD)
    lane = lax.broadcasted_iota(jnp.int32, shape, 1)
    _used = (_COL_H, _COL_C, _COL_L, _COL_LC, _COL_OO, _COL_OL,
             _COL_OLC, _COL_F, _COL_STD, _COL_OV)
    onehot = {j: (lane == j).astype(jnp.float32) for j in _used}

    def step(c0, u1, u2, ol):
        cpos = c0 > 0.0
        t = jnp.tanh(koo_h + c0 * aoo_h)
        r = pl.reciprocal(c0, approx=True)
        a1 = hoo1 * t
        olc = jnp.where(cpos, jnp.minimum(ol, u2 * r), ol)
        f = (g1 - a1) - olc
        d = c0 - thr
        ad = jnp.abs(d)
        c1 = (f * c0 + u1) - jnp.minimum(sig * d, f * ad)
        return c1, a1, olc, f

    r_start = max(0, time_lag - 0) if block_rows == batch else \
        jnp.maximum(0, time_lag - base)
    r_end = block_rows if block_rows == batch else \
        jnp.minimum(block_rows, batch - base)

    if block_rows == batch:
        if time_lag > 0:
            out_ref[pl.ds(0, time_lag), :] = jnp.zeros(
                (time_lag, _LANES), out_ref.dtype)
    else:
        @pl.when(r_start > 0)
        def _():
            out_ref[...] = jnp.zeros(out_ref.shape, out_ref.dtype)

    def row_body(r, c0):
        row_off = (base + r) * seq_len
        c = c0
        for t in range(seq_len - 1):
            c = step(c, u1_ref[row_off + t], u2_ref[row_off + t],
                     ol_ref[row_off + t])[0]
        idx = row_off + seq_len - 1
        ol = ol_ref[idx]
        c_new, a1, olc, f = step(c, u1_ref[idx], u2_ref[idx], ol)
        oo = hoo1 + a1
        sgn = jnp.sign(c * _INV_SCALE_MR - exp_yrm)
        ov = jnp.minimum(sig * sgn, f)
        packed = ((oo * c) * onehot[_COL_H]
                  + c * onehot[_COL_C]
                  + (ol * c) * onehot[_COL_L]
                  + (olc * c) * onehot[_COL_LC]
                  + oo * onehot[_COL_OO]
                  + ol * onehot[_COL_OL]
                  + olc * onehot[_COL_OLC]
                  + f * onehot[_COL_F]
                  + obs_std * onehot[_COL_STD]
                  + ov * onehot[_COL_OV])
        out_ref[pl.ds(r, 1), :] = packed
        return c_new

    if block_rows == batch:
        c_final = lax.fori_loop(r_start, r_end, row_body, c_state[...],
                                unroll=2)
    else:
        c_final = lax.fori_loop(r_start, r_end, row_body, c_state[...])
    c_state[...] = c_final


def _forward(x, y_obs, params, p_mean, p_std, *, time_lag, spin_len,
             train_len):
    batch, seq, _ = x.shape
    x = x.astype(jnp.float32)
    u1 = x[:, :, 0].reshape(-1)
    u2 = x[:, :, 1].reshape(-1)

    f32 = lambda v: jnp.asarray(v, jnp.float32).reshape(())
    w_r_yom = f32(params['weight_r_yom'])
    w_r_ylm = f32(params['weight_r_ylm'])
    w_r_yfm = f32(params['weight_r_yfm'])
    w_r_yvm = f32(params['weight_r_yvm'])
    b0_yom = f32(params['bias_b0_yom'])
    w_b1_yom = f32(params['weight_b1_yom'])
    b0_ylm = f32(params['bias_b0_ylm'])
    w_b2_ylm = f32(params['weight_b2_ylm'])
    b0_yrm = f32(params['bias_b0_yrm'])
    mo = f32(p_mean)
    so = f32(p_std)

    e_o, e_l, e_f = jnp.exp(w_r_yom), jnp.exp(w_r_ylm), jnp.exp(w_r_yfm)
    denom = e_o + e_l + e_f
    oo1 = e_o / denom
    ol1 = e_l / denom
    sig_yvm = jax.nn.sigmoid(w_r_yvm)
    exp_yrm = jnp.exp(b0_yrm)
    thr = exp_yrm * jnp.float32(_SCALE_MR)
    a_oo = w_b1_yom / so
    k_oo = b0_yom - mo * a_oo
    a_ol = w_b2_ylm / jnp.float32(_SL)
    k_ol = b0_ylm - jnp.float32(_ML) * a_ol
    ol_all = (ol1 * jax.nn.sigmoid(k_ol + u2 * a_ol)).astype(jnp.float32)
    obs_std = jnp.std(y_obs[spin_len:train_len].astype(jnp.float32), ddof=1)

    hoo1 = 0.5 * oo1
    p_vec = jnp.stack([hoo1, 1.0 - hoo1, 0.5 * k_oo, 0.5 * a_oo, sig_yvm,
                       exp_yrm, thr, obs_std]).astype(jnp.float32)

    block_rows = _round_up(batch, 8)
    grid = pl.cdiv(batch, block_rows)

    _kernel_fn = functools.partial(_rnn_kernel, batch=batch, seq_len=seq,
                                   time_lag=time_lag, block_rows=block_rows)

    out = pl.pallas_call(
        _kernel_fn,
        out_shape=jax.ShapeDtypeStruct((batch, _LANES), jnp.float32),
        grid_spec=pltpu.PrefetchScalarGridSpec(
            num_scalar_prefetch=0,
            grid=(grid,),
            in_specs=[
                pl.BlockSpec(memory_space=pltpu.MemorySpace.SMEM),
                pl.BlockSpec(memory_space=pltpu.MemorySpace.SMEM),
                pl.BlockSpec(memory_space=pltpu.MemorySpace.SMEM),
                pl.BlockSpec(memory_space=pltpu.MemorySpace.SMEM),
            ],
            out_specs=pl.BlockSpec((block_rows, _LANES), lambda i: (i, 0)),
            scratch_shapes=[pltpu.VMEM((1, _LANES), jnp.float32)],
        ),
        compiler_params=pltpu.CompilerParams(
            dimension_semantics=("arbitrary",)),
    )(u1, u2, ol_all, p_vec)

    col = lambda j: out[:, j:j + 1]
    h_n = col(_COL_H)
    obs_std_col = col(_COL_STD)
    h_nout = jnp.concatenate([h_n, obs_std_col], axis=1)
    return (h_n, col(_COL_C), col(_COL_L), col(_COL_LC), col(_COL_BP),
            col(_COL_IB), col(_COL_OO), col(_COL_OL), col(_COL_OLC),
            col(_COL_F), h_nout, obs_std_col, col(_COL_OV))


def kernel(x, y_obs, weight_r_yom, weight_r_ylm, weight_r_yfm, weight_r_yvm,
           bias_b0_yom, weight_b1_yom, bias_b0_ylm, weight_b2_ylm,
           bias_b0_yrm, p_mean, p_std):
    params = {
        'weight_r_yom': weight_r_yom,
        'weight_r_ylm': weight_r_ylm,
        'weight_r_yfm': weight_r_yfm,
        'weight_r_yvm': weight_r_yvm,
        'bias_b0_yom': bias_b0_yom,
        'weight_b1_yom': weight_b1_yom,
        'bias_b0_ylm': bias_b0_ylm,
        'weight_b2_ylm': weight_b2_ylm,
        'bias_b0_yrm': bias_b0_yrm,
    }
    return _forward(x, y_obs, params, p_mean, p_std,
                    time_lag=128, spin_len=128, train_len=4096)


# speedup vs baseline: 5.0673x; 1.0008x over previous
import functools

import jax
import jax.numpy as jnp
from jax import lax
from jax.experimental import pallas as pl
from jax.experimental.pallas import tpu as pltpu

_ML = 2.9086
_SL = 1.898
_SCALE_MR = 500.0
_INV_SCALE_MR = 1.0 / _SCALE_MR
_LANES = 128

_COL_H = 0
_COL_C = 1
_COL_L = 2
_COL_LC = 3
_COL_BP = 4
_COL_IB = 5
_COL_OO = 6
_COL_OL = 7
_COL_OLC = 8
_COL_F = 9
_COL_STD = 10
_COL_OV = 11

(_P_HOO1, _P_G1, _P_KOOH, _P_AOOH, _P_SIG, _P_EXP, _P_THR, _P_STD) = range(8)
_N_PARAMS = 8


def _round_up(x, m):
    return (x + m - 1) // m * m


def _rnn_kernel(u1_ref, u2_ref, ol_ref, p_ref, out_ref, c_state, *,
                batch, seq_len, time_lag, block_rows):
    blk = pl.program_id(0)
    base = blk * block_rows

    @pl.when(blk == 0)
    def _():
        c_state[...] = jnp.zeros_like(c_state)

    shape = (1, _LANES)

    def splat(j):
        return jnp.full(shape, p_ref[j], dtype=jnp.float32)

    hoo1 = splat(_P_HOO1)
    g1 = splat(_P_G1)
    koo_h = splat(_P_KOOH)
    aoo_h = splat(_P_AOOH)
    sig = splat(_P_SIG)
    exp_yrm = splat(_P_EXP)
    thr = splat(_P_THR)
    obs_std = splat(_P_STD)
    lane = lax.broadcasted_iota(jnp.int32, shape, 1)
    _used = (_COL_H, _COL_C, _COL_L, _COL_LC, _COL_OO, _COL_OL,
             _COL_OLC, _COL_F, _COL_STD, _COL_OV)
    onehot = {j: (lane == j).astype(jnp.float32) for j in _used}

    def step(c0, u1, u2, ol):
        cpos = c0 > 0.0
        t = jnp.tanh(koo_h + c0 * aoo_h)
        r = pl.reciprocal(c0, approx=True)
        a1 = hoo1 * t
        w = g1 - a1
        d = c0 - thr
        ad = jnp.abs(d)
        cm = c0 - ad
        e = u1 - sig * d
        q = ol * c0
        olc_c0 = jnp.where(cpos, jnp.minimum(q, u2), q)
        olc = jnp.where(cpos, jnp.minimum(ol, u2 * r), ol)
        c1a = (w * c0 + e) - olc_c0
        c1b = (w * cm + u1) - olc * cm
        c1 = jnp.maximum(c1a, c1b)
        return c1, a1, olc, q, olc_c0

    r_start = max(0, time_lag - 0) if block_rows == batch else \
        jnp.maximum(0, time_lag - base)
    r_end = block_rows if block_rows == batch else \
        jnp.minimum(block_rows, batch - base)

    if block_rows == batch:
        if time_lag > 0:
            out_ref[pl.ds(0, time_lag), :] = jnp.zeros(
                (time_lag, _LANES), out_ref.dtype)
    else:
        @pl.when(r_start > 0)
        def _():
            out_ref[...] = jnp.zeros(out_ref.shape, out_ref.dtype)

    def row_body(r, c0):
        row_off = (base + r) * seq_len
        c = c0
        for t in range(seq_len - 1):
            c = step(c, u1_ref[row_off + t], u2_ref[row_off + t],
                     ol_ref[row_off + t])[0]
        idx = row_off + seq_len - 1
        ol = ol_ref[idx]
        c_new, a1, olc, q, olc_c0 = step(c, u1_ref[idx], u2_ref[idx], ol)
        oo = hoo1 + a1
        f = (g1 - a1) - olc
        sgn = jnp.sign(c * _INV_SCALE_MR - exp_yrm)
        ov = jnp.minimum(sig * sgn, f)
        packed = ((oo * c) * onehot[_COL_H]
                  + c * onehot[_COL_C]
                  + q * onehot[_COL_L]
                  + olc_c0 * onehot[_COL_LC]
                  + oo * onehot[_COL_OO]
                  + ol * onehot[_COL_OL]
                  + olc * onehot[_COL_OLC]
                  + f * onehot[_COL_F]
                  + obs_std * onehot[_COL_STD]
                  + ov * onehot[_COL_OV])
        out_ref[pl.ds(r, 1), :] = packed
        return c_new

    if block_rows == batch:
        c_final = lax.fori_loop(r_start, r_end, row_body, c_state[...],
                                unroll=2)
    else:
        c_final = lax.fori_loop(r_start, r_end, row_body, c_state[...])
    c_state[...] = c_final


def _forward(x, y_obs, params, p_mean, p_std, *, time_lag, spin_len,
             train_len):
    batch, seq, _ = x.shape
    x = x.astype(jnp.float32)
    u1 = x[:, :, 0].reshape(-1)
    u2 = x[:, :, 1].reshape(-1)

    f32 = lambda v: jnp.asarray(v, jnp.float32).reshape(())
    w_r_yom = f32(params['weight_r_yom'])
    w_r_ylm = f32(params['weight_r_ylm'])
    w_r_yfm = f32(params['weight_r_yfm'])
    w_r_yvm = f32(params['weight_r_yvm'])
    b0_yom = f32(params['bias_b0_yom'])
    w_b1_yom = f32(params['weight_b1_yom'])
    b0_ylm = f32(params['bias_b0_ylm'])
    w_b2_ylm = f32(params['weight_b2_ylm'])
    b0_yrm = f32(params['bias_b0_yrm'])
    mo = f32(p_mean)
    so = f32(p_std)

    e_o, e_l, e_f = jnp.exp(w_r_yom), jnp.exp(w_r_ylm), jnp.exp(w_r_yfm)
    denom = e_o + e_l + e_f
    oo1 = e_o / denom
    ol1 = e_l / denom
    sig_yvm = jax.nn.sigmoid(w_r_yvm)
    exp_yrm = jnp.exp(b0_yrm)
    thr = exp_yrm * jnp.float32(_SCALE_MR)
    a_oo = w_b1_yom / so
    k_oo = b0_yom - mo * a_oo
    a_ol = w_b2_ylm / jnp.float32(_SL)
    k_ol = b0_ylm - jnp.float32(_ML) * a_ol
    ol_all = (ol1 * jax.nn.sigmoid(k_ol + u2 * a_ol)).astype(jnp.float32)
    obs_std = jnp.std(y_obs[spin_len:train_len].astype(jnp.float32), ddof=1)

    hoo1 = 0.5 * oo1
    p_vec = jnp.stack([hoo1, 1.0 - hoo1, 0.5 * k_oo, 0.5 * a_oo, sig_yvm,
                       exp_yrm, thr, obs_std]).astype(jnp.float32)

    block_rows = _round_up(batch, 8)
    grid = pl.cdiv(batch, block_rows)

    _kernel_fn = functools.partial(_rnn_kernel, batch=batch, seq_len=seq,
                                   time_lag=time_lag, block_rows=block_rows)

    out = pl.pallas_call(
        _kernel_fn,
        out_shape=jax.ShapeDtypeStruct((batch, _LANES), jnp.float32),
        grid_spec=pltpu.PrefetchScalarGridSpec(
            num_scalar_prefetch=0,
            grid=(grid,),
            in_specs=[
                pl.BlockSpec(memory_space=pltpu.MemorySpace.SMEM),
                pl.BlockSpec(memory_space=pltpu.MemorySpace.SMEM),
                pl.BlockSpec(memory_space=pltpu.MemorySpace.SMEM),
                pl.BlockSpec(memory_space=pltpu.MemorySpace.SMEM),
            ],
            out_specs=pl.BlockSpec((block_rows, _LANES), lambda i: (i, 0)),
            scratch_shapes=[pltpu.VMEM((1, _LANES), jnp.float32)],
        ),
        compiler_params=pltpu.CompilerParams(
            dimension_semantics=("arbitrary",)),
    )(u1, u2, ol_all, p_vec)

    col = lambda j: out[:, j:j + 1]
    h_n = col(_COL_H)
    obs_std_col = col(_COL_STD)
    h_nout = jnp.concatenate([h_n, obs_std_col], axis=1)
    return (h_n, col(_COL_C), col(_COL_L), col(_COL_LC), col(_COL_BP),
            col(_COL_IB), col(_COL_OO), col(_COL_OL), col(_COL_OLC),
            col(_COL_F), h_nout, obs_std_col, col(_COL_OV))


def kernel(x, y_obs, weight_r_yom, weight_r_ylm, weight_r_yfm, weight_r_yvm,
           bias_b0_yom, weight_b1_yom, bias_b0_ylm, weight_b2_ylm,
           bias_b0_yrm, p_mean, p_std):
    params = {
        'weight_r_yom': weight_r_yom,
        'weight_r_ylm': weight_r_ylm,
        'weight_r_yfm': weight_r_yfm,
        'weight_r_yvm': weight_r_yvm,
        'bias_b0_yom': bias_b0_yom,
        'weight_b1_yom': weight_b1_yom,
        'bias_b0_ylm': bias_b0_ylm,
        'weight_b2_ylm': weight_b2_ylm,
        'bias_b0_yrm': bias_b0_yrm,
    }
    return _forward(x, y_obs, params, p_mean, p_std,
                    time_lag=128, spin_len=128, train_len=4096)


# speedup vs baseline: 5.7187x; 1.1286x over previous
import functools

import jax
import jax.numpy as jnp
from jax import lax
from jax.experimental import pallas as pl
from jax.experimental.pallas import tpu as pltpu

_ML = 2.9086
_SL = 1.898
_SCALE_MR = 500.0
_INV_SCALE_MR = 1.0 / _SCALE_MR
_LANES = 128

_COL_H = 0
_COL_C = 1
_COL_L = 2
_COL_LC = 3
_COL_BP = 4
_COL_IB = 5
_COL_OO = 6
_COL_OL = 7
_COL_OLC = 8
_COL_F = 9
_COL_STD = 10
_COL_OV = 11

(_P_HOO1, _P_G1, _P_KOOH, _P_AOOH, _P_SIG, _P_EXP, _P_THR, _P_STD) = range(8)
_N_PARAMS = 8


def _round_up(x, m):
    return (x + m - 1) // m * m


def _rnn_kernel(u1_ref, u2_ref, ol_ref, p_ref, out_ref, c_state, *,
                batch, seq_len, time_lag, block_rows):
    blk = pl.program_id(0)
    base = blk * block_rows

    @pl.when(blk == 0)
    def _():
        c_state[...] = jnp.zeros_like(c_state)

    shape = (1, _LANES)

    def splat(j):
        return jnp.full(shape, p_ref[j], dtype=jnp.float32)

    hoo1 = splat(_P_HOO1)
    g1 = splat(_P_G1)
    koo_h = splat(_P_KOOH)
    aoo_h = splat(_P_AOOH)
    sig = splat(_P_SIG)
    exp_yrm = splat(_P_EXP)
    thr = splat(_P_THR)
    obs_std = splat(_P_STD)
    lane = lax.broadcasted_iota(jnp.int32, shape, 1)
    _used = (_COL_H, _COL_C, _COL_L, _COL_LC, _COL_OO, _COL_OL,
             _COL_OLC, _COL_F, _COL_STD, _COL_OV)
    onehot = {j: (lane == j).astype(jnp.float32) for j in _used}

    def step(c0, u1, u2, ol):
        cpos = c0 > 0.0
        t = jnp.tanh(koo_h + c0 * aoo_h)
        r = pl.reciprocal(c0, approx=True)
        d = c0 - thr
        ad = jnp.abs(d)
        cm = c0 - ad
        e = u1 - sig * d
        q = ol * c0
        olc_c0 = jnp.where(cpos, jnp.minimum(q, u2), q)
        olc = jnp.where(cpos, jnp.minimum(ol, u2 * r), ol)
        hc = hoo1 * c0
        hcm = hoo1 * cm
        k1 = (g1 * c0 + e) - olc_c0
        k2 = g1 * cm + u1
        c1a = k1 - t * hc
        c1b = (k2 - t * hcm) - olc * cm
        c1 = jnp.maximum(c1a, c1b)
        return c1, t, olc, q, olc_c0

    r_start = max(0, time_lag - 0) if block_rows == batch else \
        jnp.maximum(0, time_lag - base)
    r_end = block_rows if block_rows == batch else \
        jnp.minimum(block_rows, batch - base)

    if block_rows == batch:
        if time_lag > 0:
            out_ref[pl.ds(0, time_lag), :] = jnp.zeros(
                (time_lag, _LANES), out_ref.dtype)
    else:
        @pl.when(r_start > 0)
        def _():
            out_ref[...] = jnp.zeros(out_ref.shape, out_ref.dtype)

    def row_body(r, c0):
        row_off = (base + r) * seq_len
        c = c0
        for t in range(seq_len - 1):
            c = step(c, u1_ref[row_off + t], u2_ref[row_off + t],
                     ol_ref[row_off + t])[0]
        idx = row_off + seq_len - 1
        ol = ol_ref[idx]
        c_new, t, olc, q, olc_c0 = step(c, u1_ref[idx], u2_ref[idx], ol)
        a1 = hoo1 * t
        oo = hoo1 + a1
        f = (g1 - a1) - olc
        sgn = jnp.sign(c * _INV_SCALE_MR - exp_yrm)
        ov = jnp.minimum(sig * sgn, f)
        packed = ((oo * c) * onehot[_COL_H]
                  + c * onehot[_COL_C]
                  + q * onehot[_COL_L]
                  + olc_c0 * onehot[_COL_LC]
                  + oo * onehot[_COL_OO]
                  + ol * onehot[_COL_OL]
                  + olc * onehot[_COL_OLC]
                  + f * onehot[_COL_F]
                  + obs_std * onehot[_COL_STD]
                  + ov * onehot[_COL_OV])
        out_ref[pl.ds(r, 1), :] = packed
        return c_new

    if block_rows == batch:
        c_final = lax.fori_loop(r_start, r_end, row_body, c_state[...],
                                unroll=2)
    else:
        c_final = lax.fori_loop(r_start, r_end, row_body, c_state[...])
    c_state[...] = c_final


def _forward(x, y_obs, params, p_mean, p_std, *, time_lag, spin_len,
             train_len):
    batch, seq, _ = x.shape
    x = x.astype(jnp.float32)
    u1 = x[:, :, 0].reshape(-1)
    u2 = x[:, :, 1].reshape(-1)

    f32 = lambda v: jnp.asarray(v, jnp.float32).reshape(())
    w_r_yom = f32(params['weight_r_yom'])
    w_r_ylm = f32(params['weight_r_ylm'])
    w_r_yfm = f32(params['weight_r_yfm'])
    w_r_yvm = f32(params['weight_r_yvm'])
    b0_yom = f32(params['bias_b0_yom'])
    w_b1_yom = f32(params['weight_b1_yom'])
    b0_ylm = f32(params['bias_b0_ylm'])
    w_b2_ylm = f32(params['weight_b2_ylm'])
    b0_yrm = f32(params['bias_b0_yrm'])
    mo = f32(p_mean)
    so = f32(p_std)

    e_o, e_l, e_f = jnp.exp(w_r_yom), jnp.exp(w_r_ylm), jnp.exp(w_r_yfm)
    denom = e_o + e_l + e_f
    oo1 = e_o / denom
    ol1 = e_l / denom
    sig_yvm = jax.nn.sigmoid(w_r_yvm)
    exp_yrm = jnp.exp(b0_yrm)
    thr = exp_yrm * jnp.float32(_SCALE_MR)
    a_oo = w_b1_yom / so
    k_oo = b0_yom - mo * a_oo
    a_ol = w_b2_ylm / jnp.float32(_SL)
    k_ol = b0_ylm - jnp.float32(_ML) * a_ol
    ol_all = (ol1 * jax.nn.sigmoid(k_ol + u2 * a_ol)).astype(jnp.float32)
    obs_std = jnp.std(y_obs[spin_len:train_len].astype(jnp.float32), ddof=1)

    hoo1 = 0.5 * oo1
    p_vec = jnp.stack([hoo1, 1.0 - hoo1, 0.5 * k_oo, 0.5 * a_oo, sig_yvm,
                       exp_yrm, thr, obs_std]).astype(jnp.float32)

    block_rows = _round_up(batch, 8)
    grid = pl.cdiv(batch, block_rows)

    _kernel_fn = functools.partial(_rnn_kernel, batch=batch, seq_len=seq,
                                   time_lag=time_lag, block_rows=block_rows)

    out = pl.pallas_call(
        _kernel_fn,
        out_shape=jax.ShapeDtypeStruct((batch, _LANES), jnp.float32),
        grid_spec=pltpu.PrefetchScalarGridSpec(
            num_scalar_prefetch=0,
            grid=(grid,),
            in_specs=[
                pl.BlockSpec(memory_space=pltpu.MemorySpace.SMEM),
                pl.BlockSpec(memory_space=pltpu.MemorySpace.SMEM),
                pl.BlockSpec(memory_space=pltpu.MemorySpace.SMEM),
                pl.BlockSpec(memory_space=pltpu.MemorySpace.SMEM),
            ],
            out_specs=pl.BlockSpec((block_rows, _LANES), lambda i: (i, 0)),
            scratch_shapes=[pltpu.VMEM((1, _LANES), jnp.float32)],
        ),
        compiler_params=pltpu.CompilerParams(
            dimension_semantics=("arbitrary",)),
    )(u1, u2, ol_all, p_vec)

    col = lambda j: out[:, j:j + 1]
    h_n = col(_COL_H)
    obs_std_col = col(_COL_STD)
    h_nout = jnp.concatenate([h_n, obs_std_col], axis=1)
    return (h_n, col(_COL_C), col(_COL_L), col(_COL_LC), col(_COL_BP),
            col(_COL_IB), col(_COL_OO), col(_COL_OL), col(_COL_OLC),
            col(_COL_F), h_nout, obs_std_col, col(_COL_OV))


def kernel(x, y_obs, weight_r_yom, weight_r_ylm, weight_r_yfm, weight_r_yvm,
           bias_b0_yom, weight_b1_yom, bias_b0_ylm, weight_b2_ylm,
           bias_b0_yrm, p_mean, p_std):
    params = {
        'weight_r_yom': weight_r_yom,
        'weight_r_ylm': weight_r_ylm,
        'weight_r_yfm': weight_r_yfm,
        'weight_r_yvm': weight_r_yvm,
        'bias_b0_yom': bias_b0_yom,
        'weight_b1_yom': weight_b1_yom,
        'bias_b0_ylm': bias_b0_ylm,
        'weight_b2_ylm': weight_b2_ylm,
        'bias_b0_yrm': bias_b0_yrm,
    }
    return _forward(x, y_obs, params, p_mean, p_std,
                    time_lag=128, spin_len=128, train_len=4096)


# speedup vs baseline: 6.1744x; 1.0797x over previous
import functools

import jax
import jax.numpy as jnp
from jax import lax
from jax.experimental import pallas as pl
from jax.experimental.pallas import tpu as pltpu

_ML = 2.9086
_SL = 1.898
_SCALE_MR = 500.0
_INV_SCALE_MR = 1.0 / _SCALE_MR
_LANES = 128

_COL_H = 0
_COL_C = 1
_COL_L = 2
_COL_LC = 3
_COL_BP = 4
_COL_IB = 5
_COL_OO = 6
_COL_OL = 7
_COL_OLC = 8
_COL_F = 9
_COL_STD = 10
_COL_OV = 11

(_P_HOO1, _P_G1, _P_KOOH, _P_AOOH, _P_SIG, _P_EXP, _P_THR, _P_STD) = range(8)
_N_PARAMS = 8

_GROUP = 128


def _round_up(x, m):
    return (x + m - 1) // m * m


def _rnn_kernel(u1_ref, u2_ref, ol_ref, p_ref, out_ref, c_state, scr_ref, *,
                batch, seq_len, time_lag):
    c_state[...] = jnp.zeros_like(c_state)

    shape = (1, _LANES)

    def splat(j):
        return jnp.full(shape, p_ref[j], dtype=jnp.float32)

    hoo1 = splat(_P_HOO1)
    g1 = splat(_P_G1)
    koo_h = splat(_P_KOOH)
    aoo_h = splat(_P_AOOH)
    sig = splat(_P_SIG)
    exp_yrm = splat(_P_EXP)
    thr = splat(_P_THR)
    obs_std = splat(_P_STD)
    lane = lax.broadcasted_iota(jnp.int32, shape, 1)
    _used = (_COL_H, _COL_C, _COL_L, _COL_LC, _COL_OO, _COL_OL,
             _COL_OLC, _COL_F, _COL_STD, _COL_OV)
    onehot = {j: (lane == j).astype(jnp.float32) for j in _used}

    def step(c0, u1, u2, ol):
        cpos = c0 > 0.0
        t = jnp.tanh(koo_h + c0 * aoo_h)
        r = pl.reciprocal(c0, approx=True)
        d = c0 - thr
        ad = jnp.abs(d)
        cm = c0 - ad
        e = u1 - sig * d
        q = ol * c0
        olc_c0 = jnp.where(cpos, jnp.minimum(q, u2), q)
        olc = jnp.where(cpos, jnp.minimum(ol, u2 * r), ol)
        hc = hoo1 * c0
        hcm = hoo1 * cm
        k1 = (g1 * c0 + e) - olc_c0
        k2 = g1 * cm + u1
        c1a = k1 - t * hc
        c1b = (k2 - t * hcm) - olc * cm
        c1 = jnp.maximum(c1a, c1b)
        return c1, t, olc, q, olc_c0

    out_ref[:, pl.ds(0, time_lag)] = jnp.zeros((_LANES, time_lag),
                                               out_ref.dtype)

    def row_body(g, k, c0):
        row_off = (g * _GROUP + k) * seq_len
        c = c0
        for t in range(seq_len - 1):
            c = step(c, u1_ref[row_off + t], u2_ref[row_off + t],
                     ol_ref[row_off + t])[0]
        idx = row_off + seq_len - 1
        ol = ol_ref[idx]
        c_new, t, olc, q, olc_c0 = step(c, u1_ref[idx], u2_ref[idx], ol)
        a1 = hoo1 * t
        oo = hoo1 + a1
        f = (g1 - a1) - olc
        sgn = jnp.sign(c * _INV_SCALE_MR - exp_yrm)
        ov = jnp.minimum(sig * sgn, f)
        packed = ((oo * c) * onehot[_COL_H]
                  + c * onehot[_COL_C]
                  + q * onehot[_COL_L]
                  + olc_c0 * onehot[_COL_LC]
                  + oo * onehot[_COL_OO]
                  + ol * onehot[_COL_OL]
                  + olc * onehot[_COL_OLC]
                  + f * onehot[_COL_F]
                  + obs_std * onehot[_COL_STD]
                  + ov * onehot[_COL_OV])
        scr_ref[pl.ds(k, 1), :] = packed
        return c_new

    def group_body(g, c0):
        c1 = lax.fori_loop(0, _GROUP,
                           lambda k, c: row_body(g, k, c), c0, unroll=2)
        out_ref[:, pl.ds(g * _GROUP, _GROUP)] = scr_ref[...].T
        return c1

    c_final = lax.fori_loop(time_lag // _GROUP, batch // _GROUP,
                            group_body, c_state[...])
    c_state[...] = c_final


def _forward(x, y_obs, params, p_mean, p_std, *, time_lag, spin_len,
             train_len):
    batch, seq, _ = x.shape
    x = x.astype(jnp.float32)
    u1 = x[:, :, 0].reshape(-1)
    u2 = x[:, :, 1].reshape(-1)

    f32 = lambda v: jnp.asarray(v, jnp.float32).reshape(())
    w_r_yom = f32(params['weight_r_yom'])
    w_r_ylm = f32(params['weight_r_ylm'])
    w_r_yfm = f32(params['weight_r_yfm'])
    w_r_yvm = f32(params['weight_r_yvm'])
    b0_yom = f32(params['bias_b0_yom'])
    w_b1_yom = f32(params['weight_b1_yom'])
    b0_ylm = f32(params['bias_b0_ylm'])
    w_b2_ylm = f32(params['weight_b2_ylm'])
    b0_yrm = f32(params['bias_b0_yrm'])
    mo = f32(p_mean)
    so = f32(p_std)

    e_o, e_l, e_f = jnp.exp(w_r_yom), jnp.exp(w_r_ylm), jnp.exp(w_r_yfm)
    denom = e_o + e_l + e_f
    oo1 = e_o / denom
    ol1 = e_l / denom
    sig_yvm = jax.nn.sigmoid(w_r_yvm)
    exp_yrm = jnp.exp(b0_yrm)
    thr = exp_yrm * jnp.float32(_SCALE_MR)
    a_oo = w_b1_yom / so
    k_oo = b0_yom - mo * a_oo
    a_ol = w_b2_ylm / jnp.float32(_SL)
    k_ol = b0_ylm - jnp.float32(_ML) * a_ol
    ol_all = (ol1 * jax.nn.sigmoid(k_ol + u2 * a_ol)).astype(jnp.float32)
    obs_std = jnp.std(y_obs[spin_len:train_len].astype(jnp.float32), ddof=1)

    hoo1 = 0.5 * oo1
    p_vec = jnp.stack([hoo1, 1.0 - hoo1, 0.5 * k_oo, 0.5 * a_oo, sig_yvm,
                       exp_yrm, thr, obs_std]).astype(jnp.float32)

    _kernel_fn = functools.partial(_rnn_kernel, batch=batch, seq_len=seq,
                                   time_lag=time_lag)

    out = pl.pallas_call(
        _kernel_fn,
        out_shape=jax.ShapeDtypeStruct((_LANES, batch), jnp.float32),
        grid_spec=pltpu.PrefetchScalarGridSpec(
            num_scalar_prefetch=0,
            grid=(1,),
            in_specs=[
                pl.BlockSpec(memory_space=pltpu.MemorySpace.SMEM),
                pl.BlockSpec(memory_space=pltpu.MemorySpace.SMEM),
                pl.BlockSpec(memory_space=pltpu.MemorySpace.SMEM),
                pl.BlockSpec(memory_space=pltpu.MemorySpace.SMEM),
            ],
            out_specs=pl.BlockSpec((_LANES, batch), lambda i: (0, 0)),
            scratch_shapes=[pltpu.VMEM((1, _LANES), jnp.float32),
                            pltpu.VMEM((_GROUP, _LANES), jnp.float32)],
        ),
        compiler_params=pltpu.CompilerParams(
            dimension_semantics=("arbitrary",)),
    )(u1, u2, ol_all, p_vec)

    col = lambda j: out[j].reshape(batch, 1)
    h_n = col(_COL_H)
    obs_std_col = col(_COL_STD)
    h_nout = jnp.concatenate([h_n, obs_std_col], axis=1)
    return (h_n, col(_COL_C), col(_COL_L), col(_COL_LC), col(_COL_BP),
            col(_COL_IB), col(_COL_OO), col(_COL_OL), col(_COL_OLC),
            col(_COL_F), h_nout, obs_std_col, col(_COL_OV))


def kernel(x, y_obs, weight_r_yom, weight_r_ylm, weight_r_yfm, weight_r_yvm,
           bias_b0_yom, weight_b1_yom, bias_b0_ylm, weight_b2_ylm,
           bias_b0_yrm, p_mean, p_std):
    params = {
        'weight_r_yom': weight_r_yom,
        'weight_r_ylm': weight_r_ylm,
        'weight_r_yfm': weight_r_yfm,
        'weight_r_yvm': weight_r_yvm,
        'bias_b0_yom': bias_b0_yom,
        'weight_b1_yom': weight_b1_yom,
        'bias_b0_ylm': bias_b0_ylm,
        'weight_b2_ylm': weight_b2_ylm,
        'bias_b0_yrm': bias_b0_yrm,
    }
    return _forward(x, y_obs, params, p_mean, p_std,
                    time_lag=128, spin_len=128, train_len=4096)


# speedup vs baseline: 6.3210x; 1.0237x over previous
import functools

import jax
import jax.numpy as jnp
from jax import lax
from jax.experimental import pallas as pl
from jax.experimental.pallas import tpu as pltpu

_ML = 2.9086
_SL = 1.898
_SCALE_MR = 500.0
_INV_SCALE_MR = 1.0 / _SCALE_MR
_LANES = 128

_COL_H = 0
_COL_C = 1
_COL_L = 2
_COL_LC = 3
_COL_BP = 4
_COL_IB = 5
_COL_OO = 6
_COL_OL = 7
_COL_OLC = 8
_COL_F = 9
_COL_STD = 10
_COL_OV = 11

(_P_HOO1, _P_G1, _P_KOOH, _P_AOOH, _P_SIG, _P_EXP, _P_THR, _P_STD) = range(8)
_N_PARAMS = 8

_GROUP = 128


def _round_up(x, m):
    return (x + m - 1) // m * m


def _extract_kernel(x_ref, p2_ref, u1_ref, u2_ref, ol_ref):
    rows = x_ref.shape[0]
    ol1 = jnp.full((1, _LANES), p2_ref[0], dtype=jnp.float32)
    k_ol = jnp.full((1, _LANES), p2_ref[1], dtype=jnp.float32)
    a_ol = jnp.full((1, _LANES), p2_ref[2], dtype=jnp.float32)
    for g in range(rows // _LANES):
        t = x_ref[pl.ds(g * _LANES, _LANES), :].T
        u2r = t[1:2, :]
        u1_ref[pl.ds(g, 1), :] = t[0:1, :]
        u2_ref[pl.ds(g, 1), :] = u2r
        ol_ref[pl.ds(g, 1), :] = ol1 * jax.nn.sigmoid(k_ol + u2r * a_ol)


def _rnn_kernel(u1_ref, u2_ref, ol_ref, p_ref, out_ref, c_state, scr_ref, *,
                batch, seq_len, time_lag):
    c_state[...] = jnp.zeros_like(c_state)

    shape = (1, _LANES)

    def splat(j):
        return jnp.full(shape, p_ref[j], dtype=jnp.float32)

    hoo1 = splat(_P_HOO1)
    g1 = splat(_P_G1)
    koo_h = splat(_P_KOOH)
    aoo_h = splat(_P_AOOH)
    sig = splat(_P_SIG)
    exp_yrm = splat(_P_EXP)
    thr = splat(_P_THR)
    obs_std = splat(_P_STD)
    lane = lax.broadcasted_iota(jnp.int32, shape, 1)
    _used = (_COL_H, _COL_C, _COL_L, _COL_LC, _COL_OO, _COL_OL,
             _COL_OLC, _COL_F, _COL_STD, _COL_OV)
    onehot = {j: (lane == j).astype(jnp.float32) for j in _used}

    def step(c0, u1, u2, ol):
        cpos = c0 > 0.0
        t = jnp.tanh(koo_h + c0 * aoo_h)
        r = pl.reciprocal(c0, approx=True)
        d = c0 - thr
        ad = jnp.abs(d)
        cm = c0 - ad
        e = u1 - sig * d
        q = ol * c0
        olc_c0 = jnp.where(cpos, jnp.minimum(q, u2), q)
        olc = jnp.where(cpos, jnp.minimum(ol, u2 * r), ol)
        hc = hoo1 * c0
        hcm = hoo1 * cm
        k1 = (g1 * c0 + e) - olc_c0
        k2 = g1 * cm + u1
        c1a = k1 - t * hc
        c1b = (k2 - t * hcm) - olc * cm
        c1 = jnp.maximum(c1a, c1b)
        return c1, t, olc, q, olc_c0

    out_ref[:, pl.ds(0, time_lag)] = jnp.zeros((_LANES, time_lag),
                                               out_ref.dtype)

    def row_body(g, k, c0):
        row_off = (g * _GROUP + k) * seq_len
        c = c0
        for t in range(seq_len - 1):
            c = step(c, u1_ref[row_off + t], u2_ref[row_off + t],
                     ol_ref[row_off + t])[0]
        idx = row_off + seq_len - 1
        ol = ol_ref[idx]
        c_new, t, olc, q, olc_c0 = step(c, u1_ref[idx], u2_ref[idx], ol)
        a1 = hoo1 * t
        oo = hoo1 + a1
        f = (g1 - a1) - olc
        sgn = jnp.sign(c * _INV_SCALE_MR - exp_yrm)
        ov = jnp.minimum(sig * sgn, f)
        packed = ((oo * c) * onehot[_COL_H]
                  + c * onehot[_COL_C]
                  + q * onehot[_COL_L]
                  + olc_c0 * onehot[_COL_LC]
                  + oo * onehot[_COL_OO]
                  + ol * onehot[_COL_OL]
                  + olc * onehot[_COL_OLC]
                  + f * onehot[_COL_F]
                  + obs_std * onehot[_COL_STD]
                  + ov * onehot[_COL_OV])
        scr_ref[pl.ds(k, 1), :] = packed
        return c_new

    def group_body(g, c0):
        c1 = lax.fori_loop(0, _GROUP,
                           lambda k, c: row_body(g, k, c), c0, unroll=2)
        out_ref[:, pl.ds(g * _GROUP, _GROUP)] = scr_ref[...].T
        return c1

    c_final = lax.fori_loop(time_lag // _GROUP, batch // _GROUP,
                            group_body, c_state[...])
    c_state[...] = c_final


def _forward(x, y_obs, params, p_mean, p_std, *, time_lag, spin_len,
             train_len):
    batch, seq, _ = x.shape
    x = x.astype(jnp.float32)

    f32 = lambda v: jnp.asarray(v, jnp.float32).reshape(())
    w_r_yom = f32(params['weight_r_yom'])
    w_r_ylm = f32(params['weight_r_ylm'])
    w_r_yfm = f32(params['weight_r_yfm'])
    w_r_yvm = f32(params['weight_r_yvm'])
    b0_yom = f32(params['bias_b0_yom'])
    w_b1_yom = f32(params['weight_b1_yom'])
    b0_ylm = f32(params['bias_b0_ylm'])
    w_b2_ylm = f32(params['weight_b2_ylm'])
    b0_yrm = f32(params['bias_b0_yrm'])
    mo = f32(p_mean)
    so = f32(p_std)

    e_o, e_l, e_f = jnp.exp(w_r_yom), jnp.exp(w_r_ylm), jnp.exp(w_r_yfm)
    denom = e_o + e_l + e_f
    oo1 = e_o / denom
    ol1 = e_l / denom
    sig_yvm = jax.nn.sigmoid(w_r_yvm)
    exp_yrm = jnp.exp(b0_yrm)
    thr = exp_yrm * jnp.float32(_SCALE_MR)
    a_oo = w_b1_yom / so
    k_oo = b0_yom - mo * a_oo
    a_ol = w_b2_ylm / jnp.float32(_SL)
    k_ol = b0_ylm - jnp.float32(_ML) * a_ol
    obs_std = jnp.std(y_obs[spin_len:train_len].astype(jnp.float32), ddof=1)

    n = batch * seq
    x2 = x.reshape(n, _LANES)
    p2_vec = jnp.stack([ol1, k_ol, a_ol]).astype(jnp.float32)
    npar = 2
    nblk = max(1, n // (npar * 2048))
    rows_blk = n // (npar * nblk)
    u1_2d, u2_2d, ol_2d = pl.pallas_call(
        _extract_kernel,
        out_shape=[jax.ShapeDtypeStruct((n // _LANES, _LANES), jnp.float32)
                   for _ in range(3)],
        grid_spec=pltpu.PrefetchScalarGridSpec(
            num_scalar_prefetch=0,
            grid=(npar, nblk),
            in_specs=[
                pl.BlockSpec((rows_blk, _LANES),
                             lambda i, j: (i * nblk + j, 0)),
                pl.BlockSpec(memory_space=pltpu.MemorySpace.SMEM),
            ],
            out_specs=[
                pl.BlockSpec((rows_blk // _LANES, _LANES),
                             lambda i, j: (i * nblk + j, 0))
                for _ in range(3)
            ],
        ),
        compiler_params=pltpu.CompilerParams(
            dimension_semantics=("parallel", "arbitrary")),
    )(x2, p2_vec)
    u1 = u1_2d.reshape(-1)
    u2 = u2_2d.reshape(-1)
    ol_all = ol_2d.reshape(-1)

    hoo1 = 0.5 * oo1
    p_vec = jnp.stack([hoo1, 1.0 - hoo1, 0.5 * k_oo, 0.5 * a_oo, sig_yvm,
                       exp_yrm, thr, obs_std]).astype(jnp.float32)

    _kernel_fn = functools.partial(_rnn_kernel, batch=batch, seq_len=seq,
                                   time_lag=time_lag)

    out = pl.pallas_call(
        _kernel_fn,
        out_shape=jax.ShapeDtypeStruct((_LANES, batch), jnp.float32),
        grid_spec=pltpu.PrefetchScalarGridSpec(
            num_scalar_prefetch=0,
            grid=(1,),
            in_specs=[
                pl.BlockSpec(memory_space=pltpu.MemorySpace.SMEM),
                pl.BlockSpec(memory_space=pltpu.MemorySpace.SMEM),
                pl.BlockSpec(memory_space=pltpu.MemorySpace.SMEM),
                pl.BlockSpec(memory_space=pltpu.MemorySpace.SMEM),
            ],
            out_specs=pl.BlockSpec((_LANES, batch), lambda i: (0, 0)),
            scratch_shapes=[pltpu.VMEM((1, _LANES), jnp.float32),
                            pltpu.VMEM((_GROUP, _LANES), jnp.float32)],
        ),
        compiler_params=pltpu.CompilerParams(
            dimension_semantics=("arbitrary",)),
    )(u1, u2, ol_all, p_vec)

    col = lambda j: out[j].reshape(batch, 1)
    h_n = col(_COL_H)
    obs_std_col = col(_COL_STD)
    h_nout = jnp.concatenate([h_n, obs_std_col], axis=1)
    return (h_n, col(_COL_C), col(_COL_L), col(_COL_LC), col(_COL_BP),
            col(_COL_IB), col(_COL_OO), col(_COL_OL), col(_COL_OLC),
            col(_COL_F), h_nout, obs_std_col, col(_COL_OV))


def kernel(x, y_obs, weight_r_yom, weight_r_ylm, weight_r_yfm, weight_r_yvm,
           bias_b0_yom, weight_b1_yom, bias_b0_ylm, weight_b2_ylm,
           bias_b0_yrm, p_mean, p_std):
    params = {
        'weight_r_yom': weight_r_yom,
        'weight_r_ylm': weight_r_ylm,
        'weight_r_yfm': weight_r_yfm,
        'weight_r_yvm': weight_r_yvm,
        'bias_b0_yom': bias_b0_yom,
        'weight_b1_yom': weight_b1_yom,
        'bias_b0_ylm': bias_b0_ylm,
        'weight_b2_ylm': weight_b2_ylm,
        'bias_b0_yrm': bias_b0_yrm,
    }
    return _forward(x, y_obs, params, p_mean, p_std,
                    time_lag=128, spin_len=128, train_len=4096)


# speedup vs baseline: 6.4168x; 1.0152x over previous
import functools

import jax
import jax.numpy as jnp
from jax import lax
from jax.experimental import pallas as pl
from jax.experimental.pallas import tpu as pltpu

_ML = 2.9086
_SL = 1.898
_SCALE_MR = 500.0
_INV_SCALE_MR = 1.0 / _SCALE_MR
_LANES = 128

_COL_H = 0
_COL_C = 1
_COL_L = 2
_COL_LC = 3
_COL_BP = 4
_COL_IB = 5
_COL_OO = 6
_COL_OL = 7
_COL_OLC = 8
_COL_F = 9
_COL_STD = 10
_COL_OV = 11

(_P_HOO1, _P_G1, _P_KOOH, _P_AOOH, _P_SIG, _P_EXP, _P_THR, _P_STD) = range(8)
_N_PARAMS = 8

_GROUP = 128


def _round_up(x, m):
    return (x + m - 1) // m * m


def _extract_kernel(x_ref, p2_ref, u1_ref, u2_ref, ol_ref):
    rows = x_ref.shape[0]
    ol1 = jnp.full((1, _LANES), p2_ref[0], dtype=jnp.float32)
    k_ol = jnp.full((1, _LANES), p2_ref[1], dtype=jnp.float32)
    a_ol = jnp.full((1, _LANES), p2_ref[2], dtype=jnp.float32)
    for g in range(rows // _LANES):
        t = x_ref[pl.ds(g * _LANES, _LANES), :].T
        u2r = t[1:2, :]
        u1_ref[pl.ds(g, 1), :] = t[0:1, :]
        u2_ref[pl.ds(g, 1), :] = u2r
        ol_ref[pl.ds(g, 1), :] = ol1 * jax.nn.sigmoid(k_ol + u2r * a_ol)


def _rnn_kernel(u1_ref, u2_ref, ol_ref, p_ref, out_ref, c_state, scr_ref, *,
                batch, seq_len, time_lag):

    shape = (1, _LANES)

    def splat(j):
        return jnp.full(shape, p_ref[j], dtype=jnp.float32)

    hoo1 = splat(_P_HOO1)
    g1 = splat(_P_G1)
    koo_h = splat(_P_KOOH)
    aoo_h = splat(_P_AOOH)
    sig = splat(_P_SIG)
    exp_yrm = splat(_P_EXP)
    thr = splat(_P_THR)
    obs_std = splat(_P_STD)
    lane = lax.broadcasted_iota(jnp.int32, shape, 1)
    _used = (_COL_H, _COL_C, _COL_L, _COL_LC, _COL_OO, _COL_OL,
             _COL_OLC, _COL_F, _COL_STD, _COL_OV)
    onehot = {j: (lane == j).astype(jnp.float32) for j in _used}

    def step(c0pair, u1, u2, ol):
        c0a, c0b = c0pair
        c0 = jnp.maximum(c0a, c0b)
        cpos = c0 > 0.0
        ta = jnp.tanh(koo_h + c0a * aoo_h)
        tb = jnp.tanh(koo_h + c0b * aoo_h)
        t = jnp.where(c0a >= c0b, ta, tb)
        r = pl.reciprocal(c0, approx=True)
        d = c0 - thr
        ad = jnp.abs(d)
        cm = c0 - ad
        e = u1 - sig * d
        q = ol * c0
        olc_c0 = jnp.where(cpos, jnp.minimum(q, u2), q)
        olc = jnp.where(cpos, jnp.minimum(ol, u2 * r), ol)
        hc = hoo1 * c0
        hcm = hoo1 * cm
        k1 = (g1 * c0 + e) - olc_c0
        k2 = g1 * cm + u1
        c1a = k1 - t * hc
        c1b = (k2 - t * hcm) - olc * cm
        return (c1a, c1b), t, olc, q, olc_c0

    out_ref[:, pl.ds(0, time_lag)] = jnp.zeros((_LANES, time_lag),
                                               out_ref.dtype)

    def row_body(g, k, cp0):
        row_off = (g * _GROUP + k) * seq_len
        cp = cp0
        for t in range(seq_len - 1):
            cp = step(cp, u1_ref[row_off + t], u2_ref[row_off + t],
                      ol_ref[row_off + t])[0]
        idx = row_off + seq_len - 1
        ol = ol_ref[idx]
        c = jnp.maximum(cp[0], cp[1])
        cp_new, t, olc, q, olc_c0 = step(cp, u1_ref[idx], u2_ref[idx], ol)
        a1 = hoo1 * t
        oo = hoo1 + a1
        f = (g1 - a1) - olc
        sgn = jnp.sign(c * _INV_SCALE_MR - exp_yrm)
        ov = jnp.minimum(sig * sgn, f)
        packed = ((oo * c) * onehot[_COL_H]
                  + c * onehot[_COL_C]
                  + q * onehot[_COL_L]
                  + olc_c0 * onehot[_COL_LC]
                  + oo * onehot[_COL_OO]
                  + ol * onehot[_COL_OL]
                  + olc * onehot[_COL_OLC]
                  + f * onehot[_COL_F]
                  + obs_std * onehot[_COL_STD]
                  + ov * onehot[_COL_OV])
        scr_ref[pl.ds(k, 1), :] = packed
        return cp_new

    def group_body(g, cp0):
        cp1 = lax.fori_loop(0, _GROUP,
                            lambda k, cp: row_body(g, k, cp), cp0, unroll=2)
        out_ref[:, pl.ds(g * _GROUP, _GROUP)] = scr_ref[...].T
        return cp1

    zero = jnp.zeros(shape, jnp.float32)
    cp_final = lax.fori_loop(time_lag // _GROUP, batch // _GROUP,
                             group_body, (zero, zero))
    c_state[...] = jnp.maximum(cp_final[0], cp_final[1])


def _forward(x, y_obs, params, p_mean, p_std, *, time_lag, spin_len,
             train_len):
    batch, seq, _ = x.shape
    x = x.astype(jnp.float32)

    f32 = lambda v: jnp.asarray(v, jnp.float32).reshape(())
    w_r_yom = f32(params['weight_r_yom'])
    w_r_ylm = f32(params['weight_r_ylm'])
    w_r_yfm = f32(params['weight_r_yfm'])
    w_r_yvm = f32(params['weight_r_yvm'])
    b0_yom = f32(params['bias_b0_yom'])
    w_b1_yom = f32(params['weight_b1_yom'])
    b0_ylm = f32(params['bias_b0_ylm'])
    w_b2_ylm = f32(params['weight_b2_ylm'])
    b0_yrm = f32(params['bias_b0_yrm'])
    mo = f32(p_mean)
    so = f32(p_std)

    e_o, e_l, e_f = jnp.exp(w_r_yom), jnp.exp(w_r_ylm), jnp.exp(w_r_yfm)
    denom = e_o + e_l + e_f
    oo1 = e_o / denom
    ol1 = e_l / denom
    sig_yvm = jax.nn.sigmoid(w_r_yvm)
    exp_yrm = jnp.exp(b0_yrm)
    thr = exp_yrm * jnp.float32(_SCALE_MR)
    a_oo = w_b1_yom / so
    k_oo = b0_yom - mo * a_oo
    a_ol = w_b2_ylm / jnp.float32(_SL)
    k_ol = b0_ylm - jnp.float32(_ML) * a_ol
    obs_std = jnp.std(y_obs[spin_len:train_len].astype(jnp.float32), ddof=1)

    n = batch * seq
    x2 = x.reshape(n, _LANES)
    p2_vec = jnp.stack([ol1, k_ol, a_ol]).astype(jnp.float32)
    npar = 2
    nblk = max(1, n // (npar * 2048))
    rows_blk = n // (npar * nblk)
    u1_2d, u2_2d, ol_2d = pl.pallas_call(
        _extract_kernel,
        out_shape=[jax.ShapeDtypeStruct((n // _LANES, _LANES), jnp.float32)
                   for _ in range(3)],
        grid_spec=pltpu.PrefetchScalarGridSpec(
            num_scalar_prefetch=0,
            grid=(npar, nblk),
            in_specs=[
                pl.BlockSpec((rows_blk, _LANES),
                             lambda i, j: (i * nblk + j, 0)),
                pl.BlockSpec(memory_space=pltpu.MemorySpace.SMEM),
            ],
            out_specs=[
                pl.BlockSpec((rows_blk // _LANES, _LANES),
                             lambda i, j: (i * nblk + j, 0))
                for _ in range(3)
            ],
        ),
        compiler_params=pltpu.CompilerParams(
            dimension_semantics=("parallel", "arbitrary")),
    )(x2, p2_vec)
    u1 = u1_2d.reshape(-1)
    u2 = u2_2d.reshape(-1)
    ol_all = ol_2d.reshape(-1)

    hoo1 = 0.5 * oo1
    p_vec = jnp.stack([hoo1, 1.0 - hoo1, 0.5 * k_oo, 0.5 * a_oo, sig_yvm,
                       exp_yrm, thr, obs_std]).astype(jnp.float32)

    _kernel_fn = functools.partial(_rnn_kernel, batch=batch, seq_len=seq,
                                   time_lag=time_lag)

    out = pl.pallas_call(
        _kernel_fn,
        out_shape=jax.ShapeDtypeStruct((_LANES, batch), jnp.float32),
        grid_spec=pltpu.PrefetchScalarGridSpec(
            num_scalar_prefetch=0,
            grid=(1,),
            in_specs=[
                pl.BlockSpec(memory_space=pltpu.MemorySpace.SMEM),
                pl.BlockSpec(memory_space=pltpu.MemorySpace.SMEM),
                pl.BlockSpec(memory_space=pltpu.MemorySpace.SMEM),
                pl.BlockSpec(memory_space=pltpu.MemorySpace.SMEM),
            ],
            out_specs=pl.BlockSpec((_LANES, batch), lambda i: (0, 0)),
            scratch_shapes=[pltpu.VMEM((1, _LANES), jnp.float32),
                            pltpu.VMEM((_GROUP, _LANES), jnp.float32)],
        ),
        compiler_params=pltpu.CompilerParams(
            dimension_semantics=("arbitrary",)),
    )(u1, u2, ol_all, p_vec)

    col = lambda j: out[j].reshape(batch, 1)
    h_n = col(_COL_H)
    obs_std_col = col(_COL_STD)
    h_nout = jnp.concatenate([h_n, obs_std_col], axis=1)
    return (h_n, col(_COL_C), col(_COL_L), col(_COL_LC), col(_COL_BP),
            col(_COL_IB), col(_COL_OO), col(_COL_OL), col(_COL_OLC),
            col(_COL_F), h_nout, obs_std_col, col(_COL_OV))


def kernel(x, y_obs, weight_r_yom, weight_r_ylm, weight_r_yfm, weight_r_yvm,
           bias_b0_yom, weight_b1_yom, bias_b0_ylm, weight_b2_ylm,
           bias_b0_yrm, p_mean, p_std):
    params = {
        'weight_r_yom': weight_r_yom,
        'weight_r_ylm': weight_r_ylm,
        'weight_r_yfm': weight_r_yfm,
        'weight_r_yvm': weight_r_yvm,
        'bias_b0_yom': bias_b0_yom,
        'weight_b1_yom': weight_b1_yom,
        'bias_b0_ylm': bias_b0_ylm,
        'weight_b2_ylm': weight_b2_ylm,
        'bias_b0_yrm': bias_b0_yrm,
    }
    return _forward(x, y_obs, params, p_mean, p_std,
                    time_lag=128, spin_len=128, train_len=4096)


# speedup vs baseline: 6.4214x; 1.0007x over previous
import functools

import jax
import jax.numpy as jnp
from jax import lax
from jax.experimental import pallas as pl
from jax.experimental.pallas import tpu as pltpu

_ML = 2.9086
_SL = 1.898
_SCALE_MR = 500.0
_INV_SCALE_MR = 1.0 / _SCALE_MR
_LANES = 128

_COL_H = 0
_COL_C = 1
_COL_L = 2
_COL_LC = 3
_COL_BP = 4
_COL_IB = 5
_COL_OO = 6
_COL_OL = 7
_COL_OLC = 8
_COL_F = 9
_COL_STD = 10
_COL_OV = 11

(_P_HOO1, _P_G1, _P_KOOH, _P_AOOH, _P_SIG, _P_EXP, _P_THR, _P_STD) = range(8)
_N_PARAMS = 8

_GROUP = 128


def _round_up(x, m):
    return (x + m - 1) // m * m


def _extract_kernel(x_ref, p2_ref, u1_ref, u2_ref, ol_ref):
    rows = x_ref.shape[0]
    ol1 = jnp.full((1, _LANES), p2_ref[0], dtype=jnp.float32)
    k_ol = jnp.full((1, _LANES), p2_ref[1], dtype=jnp.float32)
    a_ol = jnp.full((1, _LANES), p2_ref[2], dtype=jnp.float32)
    for g in range(rows // _LANES):
        t = x_ref[pl.ds(g * _LANES, _LANES), 0:8].T
        u2r = t[1:2, :]
        u1_ref[pl.ds(g, 1), :] = t[0:1, :]
        u2_ref[pl.ds(g, 1), :] = u2r
        ol_ref[pl.ds(g, 1), :] = ol1 * jax.nn.sigmoid(k_ol + u2r * a_ol)


def _rnn_kernel(u1_ref, u2_ref, ol_ref, p_ref, out_ref, c_state, scr_ref, *,
                batch, seq_len, time_lag):

    shape = (1, _LANES)

    def splat(j):
        return jnp.full(shape, p_ref[j], dtype=jnp.float32)

    hoo1 = splat(_P_HOO1)
    g1 = splat(_P_G1)
    koo_h = splat(_P_KOOH)
    aoo_h = splat(_P_AOOH)
    sig = splat(_P_SIG)
    exp_yrm = splat(_P_EXP)
    thr = splat(_P_THR)
    obs_std = splat(_P_STD)
    lane = lax.broadcasted_iota(jnp.int32, shape, 1)
    _used = (_COL_H, _COL_C, _COL_L, _COL_LC, _COL_OO, _COL_OL,
             _COL_OLC, _COL_F, _COL_STD, _COL_OV)
    onehot = {j: (lane == j).astype(jnp.float32) for j in _used}

    def step(c0pair, u1, u2, ol):
        c0a, c0b = c0pair
        c0 = jnp.maximum(c0a, c0b)
        cpos = c0 > 0.0
        ta = jnp.tanh(koo_h + c0a * aoo_h)
        tb = jnp.tanh(koo_h + c0b * aoo_h)
        t = jnp.where(c0a >= c0b, ta, tb)
        r = pl.reciprocal(c0, approx=True)
        d = c0 - thr
        ad = jnp.abs(d)
        cm = c0 - ad
        e = u1 - sig * d
        q = ol * c0
        olc_c0 = jnp.where(cpos, jnp.minimum(q, u2), q)
        olc = jnp.where(cpos, jnp.minimum(ol, u2 * r), ol)
        hc = hoo1 * c0
        hcm = hoo1 * cm
        k1 = (g1 * c0 + e) - olc_c0
        k2 = g1 * cm + u1
        c1a = k1 - t * hc
        c1b = (k2 - t * hcm) - olc * cm
        return (c1a, c1b), t, olc, q, olc_c0

    out_ref[:, pl.ds(0, time_lag)] = jnp.zeros((_LANES, time_lag),
                                               out_ref.dtype)

    def row_body(g, k, cp0):
        row_off = (g * _GROUP + k) * seq_len
        cp = cp0
        for t in range(seq_len - 1):
            cp = step(cp, u1_ref[row_off + t], u2_ref[row_off + t],
                      ol_ref[row_off + t])[0]
        idx = row_off + seq_len - 1
        ol = ol_ref[idx]
        c = jnp.maximum(cp[0], cp[1])
        cp_new, t, olc, q, olc_c0 = step(cp, u1_ref[idx], u2_ref[idx], ol)
        a1 = hoo1 * t
        oo = hoo1 + a1
        f = (g1 - a1) - olc
        sgn = jnp.sign(c * _INV_SCALE_MR - exp_yrm)
        ov = jnp.minimum(sig * sgn, f)
        packed = ((oo * c) * onehot[_COL_H]
                  + c * onehot[_COL_C]
                  + q * onehot[_COL_L]
                  + olc_c0 * onehot[_COL_LC]
                  + oo * onehot[_COL_OO]
                  + ol * onehot[_COL_OL]
                  + olc * onehot[_COL_OLC]
                  + f * onehot[_COL_F]
                  + obs_std * onehot[_COL_STD]
                  + ov * onehot[_COL_OV])
        scr_ref[pl.ds(k, 1), :] = packed
        return cp_new

    def group_body(g, cp0):
        cp1 = lax.fori_loop(0, _GROUP,
                            lambda k, cp: row_body(g, k, cp), cp0, unroll=2)
        out_ref[:, pl.ds(g * _GROUP, _GROUP)] = scr_ref[...].T
        return cp1

    zero = jnp.zeros(shape, jnp.float32)
    cp_final = lax.fori_loop(time_lag // _GROUP, batch // _GROUP,
                             group_body, (zero, zero))
    c_state[...] = jnp.maximum(cp_final[0], cp_final[1])


def _forward(x, y_obs, params, p_mean, p_std, *, time_lag, spin_len,
             train_len):
    batch, seq, _ = x.shape
    x = x.astype(jnp.float32)

    f32 = lambda v: jnp.asarray(v, jnp.float32).reshape(())
    w_r_yom = f32(params['weight_r_yom'])
    w_r_ylm = f32(params['weight_r_ylm'])
    w_r_yfm = f32(params['weight_r_yfm'])
    w_r_yvm = f32(params['weight_r_yvm'])
    b0_yom = f32(params['bias_b0_yom'])
    w_b1_yom = f32(params['weight_b1_yom'])
    b0_ylm = f32(params['bias_b0_ylm'])
    w_b2_ylm = f32(params['weight_b2_ylm'])
    b0_yrm = f32(params['bias_b0_yrm'])
    mo = f32(p_mean)
    so = f32(p_std)

    e_o, e_l, e_f = jnp.exp(w_r_yom), jnp.exp(w_r_ylm), jnp.exp(w_r_yfm)
    denom = e_o + e_l + e_f
    oo1 = e_o / denom
    ol1 = e_l / denom
    sig_yvm = jax.nn.sigmoid(w_r_yvm)
    exp_yrm = jnp.exp(b0_yrm)
    thr = exp_yrm * jnp.float32(_SCALE_MR)
    a_oo = w_b1_yom / so
    k_oo = b0_yom - mo * a_oo
    a_ol = w_b2_ylm / jnp.float32(_SL)
    k_ol = b0_ylm - jnp.float32(_ML) * a_ol
    obs_std = jnp.std(y_obs[spin_len:train_len].astype(jnp.float32), ddof=1)

    n = batch * seq
    x2 = x.reshape(n, _LANES)
    p2_vec = jnp.stack([ol1, k_ol, a_ol]).astype(jnp.float32)
    npar = 2
    nblk = max(1, n // (npar * 2048))
    rows_blk = n // (npar * nblk)
    u1_2d, u2_2d, ol_2d = pl.pallas_call(
        _extract_kernel,
        out_shape=[jax.ShapeDtypeStruct((n // _LANES, _LANES), jnp.float32)
                   for _ in range(3)],
        grid_spec=pltpu.PrefetchScalarGridSpec(
            num_scalar_prefetch=0,
            grid=(npar, nblk),
            in_specs=[
                pl.BlockSpec((rows_blk, _LANES),
                             lambda i, j: (i * nblk + j, 0)),
                pl.BlockSpec(memory_space=pltpu.MemorySpace.SMEM),
            ],
            out_specs=[
                pl.BlockSpec((rows_blk // _LANES, _LANES),
                             lambda i, j: (i * nblk + j, 0))
                for _ in range(3)
            ],
        ),
        compiler_params=pltpu.CompilerParams(
            dimension_semantics=("parallel", "arbitrary")),
    )(x2, p2_vec)
    u1 = u1_2d.reshape(-1)
    u2 = u2_2d.reshape(-1)
    ol_all = ol_2d.reshape(-1)

    hoo1 = 0.5 * oo1
    p_vec = jnp.stack([hoo1, 1.0 - hoo1, 0.5 * k_oo, 0.5 * a_oo, sig_yvm,
                       exp_yrm, thr, obs_std]).astype(jnp.float32)

    _kernel_fn = functools.partial(_rnn_kernel, batch=batch, seq_len=seq,
                                   time_lag=time_lag)

    out = pl.pallas_call(
        _kernel_fn,
        out_shape=jax.ShapeDtypeStruct((_LANES, batch), jnp.float32),
        grid_spec=pltpu.PrefetchScalarGridSpec(
            num_scalar_prefetch=0,
            grid=(1,),
            in_specs=[
                pl.BlockSpec(memory_space=pltpu.MemorySpace.SMEM),
                pl.BlockSpec(memory_space=pltpu.MemorySpace.SMEM),
                pl.BlockSpec(memory_space=pltpu.MemorySpace.SMEM),
                pl.BlockSpec(memory_space=pltpu.MemorySpace.SMEM),
            ],
            out_specs=pl.BlockSpec((_LANES, batch), lambda i: (0, 0)),
            scratch_shapes=[pltpu.VMEM((1, _LANES), jnp.float32),
                            pltpu.VMEM((_GROUP, _LANES), jnp.float32)],
        ),
        compiler_params=pltpu.CompilerParams(
            dimension_semantics=("arbitrary",)),
    )(u1, u2, ol_all, p_vec)

    col = lambda j: out[j].reshape(batch, 1)
    h_n = col(_COL_H)
    obs_std_col = col(_COL_STD)
    h_nout = jnp.concatenate([h_n, obs_std_col], axis=1)
    return (h_n, col(_COL_C), col(_COL_L), col(_COL_LC), col(_COL_BP),
            col(_COL_IB), col(_COL_OO), col(_COL_OL), col(_COL_OLC),
            col(_COL_F), h_nout, obs_std_col, col(_COL_OV))


def kernel(x, y_obs, weight_r_yom, weight_r_ylm, weight_r_yfm, weight_r_yvm,
           bias_b0_yom, weight_b1_yom, bias_b0_ylm, weight_b2_ylm,
           bias_b0_yrm, p_mean, p_std):
    params = {
        'weight_r_yom': weight_r_yom,
        'weight_r_ylm': weight_r_ylm,
        'weight_r_yfm': weight_r_yfm,
        'weight_r_yvm': weight_r_yvm,
        'bias_b0_yom': bias_b0_yom,
        'weight_b1_yom': weight_b1_yom,
        'bias_b0_ylm': bias_b0_ylm,
        'weight_b2_ylm': weight_b2_ylm,
        'bias_b0_yrm': bias_b0_yrm,
    }
    return _forward(x, y_obs, params, p_mean, p_std,
                    time_lag=128, spin_len=128, train_len=4096)
